# Initial kernel scaffold; baseline (speedup 1.0000x reference)
import jax
import jax.numpy as jnp
from jax.experimental import pallas as pl


def kernel(hidden_state, gate_weight, up_W, gateproj_W, down_W, up_s, gateproj_s, down_s):
    raise NotImplementedError("write your pallas kernel here")


# speedup vs baseline: 1.4157x; 1.4157x over previous
import functools

import jax
import jax.numpy as jnp
from jax import lax
from jax.experimental import pallas as pl
from jax.experimental.pallas import tpu as pltpu
from jax.experimental.pallas import tpu_sc as plsc

_T = 2048
_D = 1024
_F = 2816
_E = 8
_R = 128
_NPAD = 4096 + _E * _R
_NT = _NPAD // _R
_SB = 512
_NC = 2
_NS = 16
_NW = _NC * _NS
_BPW = _NPAD // _NW
_GCH = 32
_TPW = _T // _NW
_CCH = 16


def _route_body(x_ref, gwt_ref, pw_ref, dest_ref, te_ref):
    x = x_ref[...]
    gwt = gwt_ref[...]
    logits = lax.dot_general(gwt, x, (((0,), (1,)), ((), ())),
                             preferred_element_type=jnp.float32)

    m1 = logits[0:1, :]
    i1 = jnp.zeros((1, _T), jnp.int32)
    for e in range(1, _E):
        c = logits[e:e + 1, :]
        upd = c > m1
        m1 = jnp.where(upd, c, m1)
        i1 = jnp.where(upd, e, i1)
    m2 = jnp.full((1, _T), -1e30, jnp.float32)
    i2 = jnp.zeros((1, _T), jnp.int32)
    for e in range(_E):
        c = logits[e:e + 1, :]
        upd = jnp.logical_and(i1 != e, c > m2)
        m2 = jnp.where(upd, c, m2)
        i2 = jnp.where(upd, e, i2)
    e2 = jnp.exp(m2 - m1)
    w1 = 1.0 / (1.0 + e2)
    w2 = e2 / (1.0 + e2)

    o1 = jnp.concatenate([(i1 == e).astype(jnp.float32) for e in range(_E)], axis=0)
    o2 = jnp.concatenate([(i2 == e).astype(jnp.float32) for e in range(_E)], axis=0)

    rr = lax.broadcasted_iota(jnp.int32, (_R, _R), 0)
    cc = lax.broadcasted_iota(jnp.int32, (_R, _R), 1)
    us = (rr < cc).astype(jnp.float32)
    carry = jnp.zeros((_E, 1), jnp.float32)
    c1_blocks = []
    for i in range(_T // _R):
        blk = lax.slice(o1, (0, i * _R), (_E, (i + 1) * _R))
        c1_blocks.append(jnp.dot(blk, us, preferred_element_type=jnp.float32) + carry)
        carry = carry + jnp.sum(blk, axis=1, keepdims=True)
    c1 = jnp.concatenate(c1_blocks, axis=1)
    c2_blocks = []
    for i in range(_T // _R):
        blk = lax.slice(o2, (0, i * _R), (_E, (i + 1) * _R))
        c2_blocks.append(jnp.dot(blk, us, preferred_element_type=jnp.float32) + carry)
        carry = carry + jnp.sum(blk, axis=1, keepdims=True)
    c2 = jnp.concatenate(c2_blocks, axis=1)

    counts = carry
    cpad = jnp.floor((counts + (_R - 1)) * (1.0 / _R)) * _R
    r8 = lax.broadcasted_iota(jnp.int32, (_E, _E), 0)
    c8 = lax.broadcasted_iota(jnp.int32, (_E, _E), 1)
    l8 = (r8 > c8).astype(jnp.float32)
    off = jnp.dot(l8, cpad, preferred_element_type=jnp.float32)
    ends = off + cpad

    rank1 = jnp.sum(o1 * c1, axis=0, keepdims=True)
    rank2 = jnp.sum(o2 * c2, axis=0, keepdims=True)
    offs1 = jnp.sum(o1 * off, axis=0, keepdims=True)
    offs2 = jnp.sum(o2 * off, axis=0, keepdims=True)
    dest1 = rank1 + offs1
    dest2 = rank2 + offs2
    dest_ref[...] = jnp.concatenate([dest1, dest2], axis=0).astype(jnp.int32)

    starts = lax.broadcasted_iota(jnp.int32, (1, 128), 1).astype(jnp.float32) * _R
    te = jnp.sum((ends <= starts).astype(jnp.int32), axis=0, keepdims=True)
    te_ref[...] = jnp.minimum(te, _E - 1)

    tok = lax.broadcasted_iota(jnp.int32, (1, _T), 1).astype(jnp.float32)
    tokpair = jnp.concatenate([tok, tok], axis=1)
    q = jnp.floor(tokpair * (1.0 / 128.0))
    r = tokpair - 128.0 * q
    wpair = jnp.concatenate([w1, w2], axis=1)
    whi = wpair.astype(jnp.bfloat16).astype(jnp.float32)
    wlo = wpair - whi
    destpair = jnp.concatenate([dest1, dest2], axis=1)
    tw = jnp.concatenate([q, r, whi, wlo], axis=0)
    for b in range(_NPAD // _SB):
        slotcol = lax.broadcasted_iota(jnp.int32, (_SB, 1), 0).astype(jnp.float32) + b * _SB
        s = (slotcol == destpair).astype(jnp.float32)
        pwb = lax.dot_general(s, tw, (((1,), (1,)), ((), ())),
                              preferred_element_type=jnp.float32)
        permb = 128.0 * pwb[:, 0:1] + pwb[:, 1:2]
        wb = pwb[:, 2:3] + pwb[:, 3:4]
        pw_ref[b * _SB:(b + 1) * _SB, :] = jnp.concatenate([permb, wb], axis=1)


def _route(x, gate_weight):
    return pl.pallas_call(
        _route_body,
        out_shape=[
            jax.ShapeDtypeStruct((_NPAD, 2), jnp.float32),
            jax.ShapeDtypeStruct((2, _T), jnp.int32),
            jax.ShapeDtypeStruct((1, 128), jnp.int32),
        ],
    )(x, gate_weight.T)


def _ffn_grouped_body(te_ref, xs_ref, w_ref, up_ref, gp_ref, dn_ref, o_ref):
    del te_ref
    xb = xs_ref[...]
    u = lax.dot_general(xb, up_ref[0], (((1,), (1,)), ((), ())),
                        preferred_element_type=jnp.float32)
    g = lax.dot_general(xb, gp_ref[0], (((1,), (1,)), ((), ())),
                        preferred_element_type=jnp.float32)
    h = u * jax.nn.sigmoid(u) * g
    h = h * w_ref[...]
    o_ref[...] = lax.dot_general(h, dn_ref[0], (((1,), (1,)), ((), ())),
                                 preferred_element_type=jnp.float32)


def _ffn_grouped(tile_eid, xs, wsrt, up_W, gateproj_W, down_W):
    grid_spec = pltpu.PrefetchScalarGridSpec(
        num_scalar_prefetch=1,
        grid=(_NT,),
        in_specs=[
            pl.BlockSpec((_R, _D), lambda i, te: (i, 0)),
            pl.BlockSpec((_R, 1), lambda i, te: (i, 0)),
            pl.BlockSpec((1, _F, _D), lambda i, te: (te[i], 0, 0),
                         pipeline_mode=pl.Buffered(buffer_count=1)),
            pl.BlockSpec((1, _F, _D), lambda i, te: (te[i], 0, 0),
                         pipeline_mode=pl.Buffered(buffer_count=1)),
            pl.BlockSpec((1, _D, _F), lambda i, te: (te[i], 0, 0),
                         pipeline_mode=pl.Buffered(buffer_count=1)),
        ],
        out_specs=pl.BlockSpec((_R, _D), lambda i, te: (i, 0)),
    )
    return pl.pallas_call(
        _ffn_grouped_body,
        grid_spec=grid_spec,
        out_shape=jax.ShapeDtypeStruct((_NPAD, _D), jnp.float32),
        compiler_params=pltpu.CompilerParams(
            dimension_semantics=("arbitrary",)),
    )(tile_eid, xs, wsrt, up_W, gateproj_W, down_W)


_SR = 256


def _ffn_shared_body(x_ref, up_ref, gp_ref, dn_ref, o_ref):
    xb = x_ref[...]
    u = lax.dot_general(xb, up_ref[...], (((1,), (1,)), ((), ())),
                        preferred_element_type=jnp.float32)
    g = lax.dot_general(xb, gp_ref[...], (((1,), (1,)), ((), ())),
                        preferred_element_type=jnp.float32)
    h = u * jax.nn.sigmoid(u) * g
    o_ref[...] = lax.dot_general(h, dn_ref[...], (((1,), (1,)), ((), ())),
                                 preferred_element_type=jnp.float32)


def _ffn_shared(x, up_s, gateproj_s, down_s):
    return pl.pallas_call(
        _ffn_shared_body,
        grid=(_T // _SR,),
        in_specs=[
            pl.BlockSpec((_SR, _D), lambda i: (i, 0)),
            pl.BlockSpec((_F, _D), lambda i: (0, 0),
                         pipeline_mode=pl.Buffered(buffer_count=1)),
            pl.BlockSpec((_F, _D), lambda i: (0, 0),
                         pipeline_mode=pl.Buffered(buffer_count=1)),
            pl.BlockSpec((_D, _F), lambda i: (0, 0),
                         pipeline_mode=pl.Buffered(buffer_count=1)),
        ],
        out_specs=pl.BlockSpec((_SR, _D), lambda i: (i, 0)),
        out_shape=jax.ShapeDtypeStruct((_T, _D), jnp.float32),
        compiler_params=pltpu.CompilerParams(
            dimension_semantics=("arbitrary",)),
    )(x, up_s, gateproj_s, down_s)


@functools.cache
def _sc_mesh():
    return plsc.VectorSubcoreMesh(core_axis_name="c", subcore_axis_name="s")


def _sc_gather_body(x_hbm, perm_hbm, xs_hbm, idx_v, rows_v, sem):
    wid = lax.axis_index("s") * _NC + lax.axis_index("c")
    base = wid * _BPW
    for c in range(_BPW // _GCH):
        pltpu.sync_copy(perm_hbm.at[pl.ds(base + c * _GCH, _GCH)], idx_v)
        pltpu.async_copy(x_hbm.at[idx_v], rows_v, sem).wait()
        pltpu.sync_copy(rows_v, xs_hbm.at[pl.ds(base + c * _GCH, _GCH)])


@functools.cache
def _sc_gather():
    return pl.kernel(
        _sc_gather_body,
        mesh=_sc_mesh(),
        out_type=jax.ShapeDtypeStruct((_NPAD, _D), jnp.float32),
        scratch_types=[
            pltpu.VMEM((_GCH,), jnp.int32),
            pltpu.VMEM((_GCH, _D), jnp.float32),
            pltpu.SemaphoreType.DMA,
        ],
    )


def _sc_combine_body(o_hbm, sh_hbm, d0_hbm, d1_hbm, y_hbm,
                     idx0, idx1, av, bv, sv, sem0, sem1):
    wid = lax.axis_index("s") * _NC + lax.axis_index("c")
    base = wid * _TPW
    for c in range(_TPW // _CCH):
        tb = base + c * _CCH
        pltpu.sync_copy(d0_hbm.at[pl.ds(tb, _CCH)], idx0)
        pltpu.sync_copy(d1_hbm.at[pl.ds(tb, _CCH)], idx1)
        cp0 = pltpu.async_copy(o_hbm.at[idx0], av, sem0)
        cp1 = pltpu.async_copy(o_hbm.at[idx1], bv, sem1)
        pltpu.sync_copy(sh_hbm.at[pl.ds(tb, _CCH)], sv)
        cp0.wait()
        cp1.wait()

        def row_fn(r, carry):
            for j in range(_D // 16):
                a = av[r, pl.ds(j * 16, 16)]
                b = bv[r, pl.ds(j * 16, 16)]
                s = sv[r, pl.ds(j * 16, 16)]
                av[r, pl.ds(j * 16, 16)] = a + b + s
            return carry

        lax.fori_loop(0, _CCH, row_fn, 0)
        pltpu.sync_copy(av, y_hbm.at[pl.ds(tb, _CCH)])


@functools.cache
def _sc_combine():
    return pl.kernel(
        _sc_combine_body,
        mesh=_sc_mesh(),
        out_type=jax.ShapeDtypeStruct((_T, _D), jnp.float32),
        scratch_types=[
            pltpu.VMEM((_CCH,), jnp.int32),
            pltpu.VMEM((_CCH,), jnp.int32),
            pltpu.VMEM((_CCH, _D), jnp.float32),
            pltpu.VMEM((_CCH, _D), jnp.float32),
            pltpu.VMEM((_CCH, _D), jnp.float32),
            pltpu.SemaphoreType.DMA,
            pltpu.SemaphoreType.DMA,
        ],
    )


def kernel(hidden_state, gate_weight, up_W, gateproj_W, down_W,
           up_s, gateproj_s, down_s):
    bsz, sl, dm = hidden_state.shape
    x = hidden_state.reshape(_T, _D)
    pw, dest, te = _route(x, gate_weight)
    perm = pw[:, 0].astype(jnp.int32)
    wsrt = pw[:, 1:2]
    d0 = dest[0]
    d1 = dest[1]
    tile_eid = te.reshape(128)[:_NT]
    xs = _sc_gather()(x, perm)
    o = _ffn_grouped(tile_eid, xs, wsrt, up_W, gateproj_W, down_W)
    sh = _ffn_shared(x, up_s, gateproj_s, down_s)
    y = _sc_combine()(o, sh, d0, d1)
    return y.reshape(bsz, sl, dm)


# speedup vs baseline: 1.4462x; 1.0215x over previous
import functools

import jax
import jax.numpy as jnp
from jax import lax
from jax.experimental import pallas as pl
from jax.experimental.pallas import tpu as pltpu
from jax.experimental.pallas import tpu_sc as plsc

_T = 2048
_D = 1024
_F = 2816
_E = 8
_R = 128
_NPAD = 4096 + _E * _R
_NT = _NPAD // _R
_SB = 512
_NC = 2
_NS = 16
_NW = _NC * _NS
_BPW = _NPAD // _NW
_GCH = 32
_TPW = _T // _NW
_CCH = 16


def _route_body(x_ref, gwt_ref, pw_ref, dest_ref, te_ref):
    x = x_ref[...]
    gwt = gwt_ref[...]
    logits = lax.dot_general(gwt, x, (((0,), (1,)), ((), ())),
                             preferred_element_type=jnp.float32)

    m1 = logits[0:1, :]
    i1 = jnp.zeros((1, _T), jnp.int32)
    for e in range(1, _E):
        c = logits[e:e + 1, :]
        upd = c > m1
        m1 = jnp.where(upd, c, m1)
        i1 = jnp.where(upd, e, i1)
    m2 = jnp.full((1, _T), -1e30, jnp.float32)
    i2 = jnp.zeros((1, _T), jnp.int32)
    for e in range(_E):
        c = logits[e:e + 1, :]
        upd = jnp.logical_and(i1 != e, c > m2)
        m2 = jnp.where(upd, c, m2)
        i2 = jnp.where(upd, e, i2)
    e2 = jnp.exp(m2 - m1)
    w1 = 1.0 / (1.0 + e2)
    w2 = e2 / (1.0 + e2)

    o1 = jnp.concatenate([(i1 == e).astype(jnp.float32) for e in range(_E)], axis=0)
    o2 = jnp.concatenate([(i2 == e).astype(jnp.float32) for e in range(_E)], axis=0)

    rr = lax.broadcasted_iota(jnp.int32, (_R, _R), 0)
    cc = lax.broadcasted_iota(jnp.int32, (_R, _R), 1)
    us = (rr < cc).astype(jnp.float32)
    carry = jnp.zeros((_E, 1), jnp.float32)
    c1_blocks = []
    for i in range(_T // _R):
        blk = lax.slice(o1, (0, i * _R), (_E, (i + 1) * _R))
        c1_blocks.append(jnp.dot(blk, us, preferred_element_type=jnp.float32) + carry)
        carry = carry + jnp.sum(blk, axis=1, keepdims=True)
    c1 = jnp.concatenate(c1_blocks, axis=1)
    c2_blocks = []
    for i in range(_T // _R):
        blk = lax.slice(o2, (0, i * _R), (_E, (i + 1) * _R))
        c2_blocks.append(jnp.dot(blk, us, preferred_element_type=jnp.float32) + carry)
        carry = carry + jnp.sum(blk, axis=1, keepdims=True)
    c2 = jnp.concatenate(c2_blocks, axis=1)

    counts = carry
    cpad = jnp.floor((counts + (_R - 1)) * (1.0 / _R)) * _R
    r8 = lax.broadcasted_iota(jnp.int32, (_E, _E), 0)
    c8 = lax.broadcasted_iota(jnp.int32, (_E, _E), 1)
    l8 = (r8 > c8).astype(jnp.float32)
    off = jnp.dot(l8, cpad, preferred_element_type=jnp.float32)
    ends = off + cpad

    rank1 = jnp.sum(o1 * c1, axis=0, keepdims=True)
    rank2 = jnp.sum(o2 * c2, axis=0, keepdims=True)
    offs1 = jnp.sum(o1 * off, axis=0, keepdims=True)
    offs2 = jnp.sum(o2 * off, axis=0, keepdims=True)
    dest1 = rank1 + offs1
    dest2 = rank2 + offs2
    dest_ref[...] = jnp.concatenate([dest1, dest2], axis=0).astype(jnp.int32)

    starts = lax.broadcasted_iota(jnp.int32, (1, 128), 1).astype(jnp.float32) * _R
    te = jnp.sum((ends <= starts).astype(jnp.int32), axis=0, keepdims=True)
    te_ref[...] = jnp.minimum(te, _E - 1)

    tok = lax.broadcasted_iota(jnp.int32, (1, _T), 1).astype(jnp.float32)
    tokpair = jnp.concatenate([tok, tok], axis=1)
    q = jnp.floor(tokpair * (1.0 / 128.0))
    r = tokpair - 128.0 * q
    wpair = jnp.concatenate([w1, w2], axis=1)
    whi = wpair.astype(jnp.bfloat16).astype(jnp.float32)
    wlo = wpair - whi
    destpair = jnp.concatenate([dest1, dest2], axis=1)
    tw = jnp.concatenate([q, r, whi, wlo], axis=0)
    for b in range(_NPAD // _SB):
        slotcol = lax.broadcasted_iota(jnp.int32, (_SB, 1), 0).astype(jnp.float32) + b * _SB
        s = (slotcol == destpair).astype(jnp.float32)
        pwb = lax.dot_general(s, tw, (((1,), (1,)), ((), ())),
                              preferred_element_type=jnp.float32)
        permb = 128.0 * pwb[:, 0:1] + pwb[:, 1:2]
        wb = pwb[:, 2:3] + pwb[:, 3:4]
        pw_ref[b * _SB:(b + 1) * _SB, :] = jnp.concatenate([permb, wb], axis=1)


def _route(x, gate_weight):
    return pl.pallas_call(
        _route_body,
        out_shape=[
            jax.ShapeDtypeStruct((_NPAD, 2), jnp.float32),
            jax.ShapeDtypeStruct((2, _T), jnp.int32),
            jax.ShapeDtypeStruct((1, 128), jnp.int32),
        ],
    )(x, gate_weight.T)


def _ffn_grouped_body(te_ref, xs_ref, w_ref, up_ref, gp_ref, dn_ref, o_ref):
    del te_ref
    xb = xs_ref[...]
    u = lax.dot_general(xb, up_ref[0], (((1,), (1,)), ((), ())),
                        preferred_element_type=jnp.float32)
    g = lax.dot_general(xb, gp_ref[0], (((1,), (1,)), ((), ())),
                        preferred_element_type=jnp.float32)
    h = u * jax.nn.sigmoid(u) * g
    h = h * w_ref[...]
    o_ref[...] = lax.dot_general(h, dn_ref[0], (((1,), (1,)), ((), ())),
                                 preferred_element_type=jnp.float32)


def _ffn_grouped(tile_eid, xs, wsrt, up_W, gateproj_W, down_W):
    grid_spec = pltpu.PrefetchScalarGridSpec(
        num_scalar_prefetch=1,
        grid=(_NT,),
        in_specs=[
            pl.BlockSpec((_R, _D), lambda i, te: (i, 0)),
            pl.BlockSpec((_R, 1), lambda i, te: (i, 0)),
            pl.BlockSpec((1, _F, _D), lambda i, te: (te[i], 0, 0),
                         pipeline_mode=pl.Buffered(buffer_count=1)),
            pl.BlockSpec((1, _F, _D), lambda i, te: (te[i], 0, 0),
                         pipeline_mode=pl.Buffered(buffer_count=1)),
            pl.BlockSpec((1, _D, _F), lambda i, te: (te[i], 0, 0),
                         pipeline_mode=pl.Buffered(buffer_count=1)),
        ],
        out_specs=pl.BlockSpec((_R, _D), lambda i, te: (i, 0)),
    )
    return pl.pallas_call(
        _ffn_grouped_body,
        grid_spec=grid_spec,
        out_shape=jax.ShapeDtypeStruct((_NPAD, _D), jnp.float32),
        compiler_params=pltpu.CompilerParams(
            dimension_semantics=("arbitrary",)),
    )(tile_eid, xs, wsrt, up_W, gateproj_W, down_W)


_SR = 256


def _ffn_shared_body(x_ref, up_ref, gp_ref, dn_ref, o_ref):
    xb = x_ref[...]
    u = lax.dot_general(xb, up_ref[...], (((1,), (1,)), ((), ())),
                        preferred_element_type=jnp.float32)
    g = lax.dot_general(xb, gp_ref[...], (((1,), (1,)), ((), ())),
                        preferred_element_type=jnp.float32)
    h = u * jax.nn.sigmoid(u) * g
    o_ref[...] = lax.dot_general(h, dn_ref[...], (((1,), (1,)), ((), ())),
                                 preferred_element_type=jnp.float32)


def _ffn_shared(x, up_s, gateproj_s, down_s):
    return pl.pallas_call(
        _ffn_shared_body,
        grid=(_T // _SR,),
        in_specs=[
            pl.BlockSpec((_SR, _D), lambda i: (i, 0)),
            pl.BlockSpec((_F, _D), lambda i: (0, 0),
                         pipeline_mode=pl.Buffered(buffer_count=1)),
            pl.BlockSpec((_F, _D), lambda i: (0, 0),
                         pipeline_mode=pl.Buffered(buffer_count=1)),
            pl.BlockSpec((_D, _F), lambda i: (0, 0),
                         pipeline_mode=pl.Buffered(buffer_count=1)),
        ],
        out_specs=pl.BlockSpec((_SR, _D), lambda i: (i, 0)),
        out_shape=jax.ShapeDtypeStruct((_T, _D), jnp.float32),
        compiler_params=pltpu.CompilerParams(
            dimension_semantics=("arbitrary",)),
    )(x, up_s, gateproj_s, down_s)


@functools.cache
def _sc_mesh():
    return plsc.VectorSubcoreMesh(core_axis_name="c", subcore_axis_name="s")


def _sc_gather_body(x_hbm, perm_hbm, xs_hbm, idx_v,
                    rows0, rows1, gs0, gs1, ws0, ws1):
    wid = lax.axis_index("s") * _NC + lax.axis_index("c")
    base = wid * _BPW
    nch = _BPW // _GCH
    bufs = (rows0, rows1)
    gsems = (gs0, gs1)
    wsems = (ws0, ws1)
    pltpu.sync_copy(perm_hbm.at[pl.ds(base, _BPW)], idx_v)
    gcp = [None] * nch
    wcp = [None] * nch
    gcp[0] = pltpu.async_copy(
        x_hbm.at[idx_v.at[pl.ds(0, _GCH)]], rows0, gs0)
    for c in range(nch):
        b = c % 2
        if c + 1 < nch:
            bn = (c + 1) % 2
            if c - 1 >= 0:
                wcp[c - 1].wait()
            gcp[c + 1] = pltpu.async_copy(
                x_hbm.at[idx_v.at[pl.ds((c + 1) * _GCH, _GCH)]],
                bufs[bn], gsems[bn])
        gcp[c].wait()
        wcp[c] = pltpu.async_copy(
            bufs[b], xs_hbm.at[pl.ds(base + c * _GCH, _GCH)], wsems[b])
    if nch >= 2:
        wcp[nch - 2].wait()
    wcp[nch - 1].wait()


@functools.cache
def _sc_gather():
    return pl.kernel(
        _sc_gather_body,
        mesh=_sc_mesh(),
        out_type=jax.ShapeDtypeStruct((_NPAD, _D), jnp.float32),
        scratch_types=[
            pltpu.VMEM((_BPW,), jnp.int32),
            pltpu.VMEM((_GCH, _D), jnp.float32),
            pltpu.VMEM((_GCH, _D), jnp.float32),
            pltpu.SemaphoreType.DMA,
            pltpu.SemaphoreType.DMA,
            pltpu.SemaphoreType.DMA,
            pltpu.SemaphoreType.DMA,
        ],
    )


def _sc_combine_body(o_hbm, sh_hbm, d0_hbm, d1_hbm, y_hbm,
                     idx0, idx1, av0, bv0, sv0, av1, bv1, sv1,
                     gsem0, gsem1, wsem0, wsem1):
    wid = lax.axis_index("s") * _NC + lax.axis_index("c")
    base = wid * _TPW
    nch = _TPW // _CCH
    avs, bvs, svs = (av0, av1), (bv0, bv1), (sv0, sv1)
    gsems = (gsem0, gsem1)
    wsems = (wsem0, wsem1)
    pltpu.sync_copy(d0_hbm.at[pl.ds(base, _TPW)], idx0)
    pltpu.sync_copy(d1_hbm.at[pl.ds(base, _TPW)], idx1)

    def issue(c):
        st = c % 2
        return [
            pltpu.async_copy(o_hbm.at[idx0.at[pl.ds(c * _CCH, _CCH)]],
                             avs[st], gsems[st]),
            pltpu.async_copy(o_hbm.at[idx1.at[pl.ds(c * _CCH, _CCH)]],
                             bvs[st], gsems[st]),
            pltpu.async_copy(sh_hbm.at[pl.ds(base + c * _CCH, _CCH)],
                             svs[st], gsems[st]),
        ]

    pend = [None] * nch
    wpend = [None] * nch
    pend[0] = issue(0)
    for c in range(nch):
        st = c % 2
        if c + 1 < nch:
            if c - 1 >= 0:
                wpend[c - 1].wait()
            pend[c + 1] = issue(c + 1)
        for cp in pend[c]:
            cp.wait()
        av, bv, sv = avs[st], bvs[st], svs[st]

        def row_fn(r, carry):
            for j in range(_D // 16):
                a = av[r, pl.ds(j * 16, 16)]
                b = bv[r, pl.ds(j * 16, 16)]
                s = sv[r, pl.ds(j * 16, 16)]
                av[r, pl.ds(j * 16, 16)] = a + b + s
            return carry

        lax.fori_loop(0, _CCH, row_fn, 0)
        wpend[c] = pltpu.async_copy(
            av, y_hbm.at[pl.ds(base + c * _CCH, _CCH)], wsems[st])
    if nch >= 2:
        wpend[nch - 2].wait()
    wpend[nch - 1].wait()


@functools.cache
def _sc_combine():
    return pl.kernel(
        _sc_combine_body,
        mesh=_sc_mesh(),
        out_type=jax.ShapeDtypeStruct((_T, _D), jnp.float32),
        scratch_types=[
            pltpu.VMEM((_TPW,), jnp.int32),
            pltpu.VMEM((_TPW,), jnp.int32),
            pltpu.VMEM((_CCH, _D), jnp.float32),
            pltpu.VMEM((_CCH, _D), jnp.float32),
            pltpu.VMEM((_CCH, _D), jnp.float32),
            pltpu.VMEM((_CCH, _D), jnp.float32),
            pltpu.VMEM((_CCH, _D), jnp.float32),
            pltpu.VMEM((_CCH, _D), jnp.float32),
            pltpu.SemaphoreType.DMA,
            pltpu.SemaphoreType.DMA,
            pltpu.SemaphoreType.DMA,
            pltpu.SemaphoreType.DMA,
        ],
    )


def kernel(hidden_state, gate_weight, up_W, gateproj_W, down_W,
           up_s, gateproj_s, down_s):
    bsz, sl, dm = hidden_state.shape
    x = hidden_state.reshape(_T, _D)
    pw, dest, te = _route(x, gate_weight)
    perm = pw[:, 0].astype(jnp.int32)
    wsrt = pw[:, 1:2]
    d0 = dest[0]
    d1 = dest[1]
    tile_eid = te.reshape(128)[:_NT]
    xs = _sc_gather()(x, perm)
    o = _ffn_grouped(tile_eid, xs, wsrt, up_W, gateproj_W, down_W)
    sh = _ffn_shared(x, up_s, gateproj_s, down_s)
    y = _sc_combine()(o, sh, d0, d1)
    return y.reshape(bsz, sl, dm)


# speedup vs baseline: 1.5553x; 1.0754x over previous
import functools

import jax
import jax.numpy as jnp
from jax import lax
from jax.experimental import pallas as pl
from jax.experimental.pallas import tpu as pltpu
from jax.experimental.pallas import tpu_sc as plsc

_T = 2048
_D = 1024
_F = 2816
_E = 8
_R = 128
_NPAD = 4096 + _E * _R
_NT = _NPAD // _R
_SB = 512
_NC = 2
_NS = 16
_NW = _NC * _NS
_BPW = _NPAD // _NW
_GCH = 32
_TPW = _T // _NW
_CCH = 16


def _route_body(x_ref, gwt_ref, pw_ref, dest_ref, te_ref):
    x = x_ref[...]
    gwt = gwt_ref[...]
    logits = lax.dot_general(gwt, x, (((0,), (1,)), ((), ())),
                             preferred_element_type=jnp.float32)

    m1 = logits[0:1, :]
    i1 = jnp.zeros((1, _T), jnp.int32)
    for e in range(1, _E):
        c = logits[e:e + 1, :]
        upd = c > m1
        m1 = jnp.where(upd, c, m1)
        i1 = jnp.where(upd, e, i1)
    m2 = jnp.full((1, _T), -1e30, jnp.float32)
    i2 = jnp.zeros((1, _T), jnp.int32)
    for e in range(_E):
        c = logits[e:e + 1, :]
        upd = jnp.logical_and(i1 != e, c > m2)
        m2 = jnp.where(upd, c, m2)
        i2 = jnp.where(upd, e, i2)
    e2 = jnp.exp(m2 - m1)
    w1 = 1.0 / (1.0 + e2)
    w2 = e2 / (1.0 + e2)

    o1 = jnp.concatenate([(i1 == e).astype(jnp.float32) for e in range(_E)], axis=0)
    o2 = jnp.concatenate([(i2 == e).astype(jnp.float32) for e in range(_E)], axis=0)

    rr = lax.broadcasted_iota(jnp.int32, (_R, _R), 0)
    cc = lax.broadcasted_iota(jnp.int32, (_R, _R), 1)
    us = (rr < cc).astype(jnp.float32)
    carry = jnp.zeros((_E, 1), jnp.float32)
    c1_blocks = []
    for i in range(_T // _R):
        blk = lax.slice(o1, (0, i * _R), (_E, (i + 1) * _R))
        c1_blocks.append(jnp.dot(blk, us, preferred_element_type=jnp.float32) + carry)
        carry = carry + jnp.sum(blk, axis=1, keepdims=True)
    c1 = jnp.concatenate(c1_blocks, axis=1)
    c2_blocks = []
    for i in range(_T // _R):
        blk = lax.slice(o2, (0, i * _R), (_E, (i + 1) * _R))
        c2_blocks.append(jnp.dot(blk, us, preferred_element_type=jnp.float32) + carry)
        carry = carry + jnp.sum(blk, axis=1, keepdims=True)
    c2 = jnp.concatenate(c2_blocks, axis=1)

    counts = carry
    cpad = jnp.floor((counts + (_R - 1)) * (1.0 / _R)) * _R
    r8 = lax.broadcasted_iota(jnp.int32, (_E, _E), 0)
    c8 = lax.broadcasted_iota(jnp.int32, (_E, _E), 1)
    l8 = (r8 > c8).astype(jnp.float32)
    off = jnp.dot(l8, cpad, preferred_element_type=jnp.float32)
    ends = off + cpad

    rank1 = jnp.sum(o1 * c1, axis=0, keepdims=True)
    rank2 = jnp.sum(o2 * c2, axis=0, keepdims=True)
    offs1 = jnp.sum(o1 * off, axis=0, keepdims=True)
    offs2 = jnp.sum(o2 * off, axis=0, keepdims=True)
    dest1 = rank1 + offs1
    dest2 = rank2 + offs2
    dest_ref[...] = jnp.concatenate([dest1, dest2], axis=0).astype(jnp.int32)

    starts = lax.broadcasted_iota(jnp.int32, (1, 128), 1).astype(jnp.float32) * _R
    te = jnp.sum((ends <= starts).astype(jnp.int32), axis=0, keepdims=True)
    te_ref[...] = jnp.minimum(te, _E - 1)

    tok = lax.broadcasted_iota(jnp.int32, (1, _T), 1).astype(jnp.float32)
    tokpair = jnp.concatenate([tok, tok], axis=1)
    q = jnp.floor(tokpair * (1.0 / 128.0))
    r = tokpair - 128.0 * q
    wpair = jnp.concatenate([w1, w2], axis=1)
    whi = wpair.astype(jnp.bfloat16).astype(jnp.float32)
    wlo = wpair - whi
    destpair = jnp.concatenate([dest1, dest2], axis=1)
    tw = jnp.concatenate([q, r, whi, wlo], axis=0)
    for b in range(_NPAD // _SB):
        slotcol = lax.broadcasted_iota(jnp.int32, (_SB, 1), 0).astype(jnp.float32) + b * _SB
        s = (slotcol == destpair).astype(jnp.float32)
        pwb = lax.dot_general(s, tw, (((1,), (1,)), ((), ())),
                              preferred_element_type=jnp.float32)
        permb = 128.0 * pwb[:, 0:1] + pwb[:, 1:2]
        wb = pwb[:, 2:3] + pwb[:, 3:4]
        pw_ref[b * _SB:(b + 1) * _SB, :] = jnp.concatenate([permb, wb], axis=1)


def _route(x, gate_weight):
    return pl.pallas_call(
        _route_body,
        out_shape=[
            jax.ShapeDtypeStruct((_NPAD, 2), jnp.float32),
            jax.ShapeDtypeStruct((2, _T), jnp.int32),
            jax.ShapeDtypeStruct((1, 128), jnp.int32),
        ],
    )(x, gate_weight.T)


def _ffn_grouped_body(te_ref, xs_ref, w_ref, up_ref, gp_ref, dn_ref, o_ref):
    del te_ref
    xb = xs_ref[...]
    u = lax.dot_general(xb, up_ref[0], (((1,), (1,)), ((), ())),
                        preferred_element_type=jnp.float32)
    g = lax.dot_general(xb, gp_ref[0], (((1,), (1,)), ((), ())),
                        preferred_element_type=jnp.float32)
    h = u * jax.nn.sigmoid(u) * g
    h = h * w_ref[...]
    o_ref[...] = lax.dot_general(h, dn_ref[0], (((1,), (1,)), ((), ())),
                                 preferred_element_type=jnp.float32)


def _ffn_grouped(tile_eid, xs, wsrt, up_W, gateproj_W, down_W):
    grid_spec = pltpu.PrefetchScalarGridSpec(
        num_scalar_prefetch=1,
        grid=(_NT,),
        in_specs=[
            pl.BlockSpec((_R, _D), lambda i, te: (i, 0)),
            pl.BlockSpec((_R, 1), lambda i, te: (i, 0)),
            pl.BlockSpec((1, _F, _D), lambda i, te: (te[i], 0, 0),
                         pipeline_mode=pl.Buffered(buffer_count=2)),
            pl.BlockSpec((1, _F, _D), lambda i, te: (te[i], 0, 0),
                         pipeline_mode=pl.Buffered(buffer_count=2)),
            pl.BlockSpec((1, _D, _F), lambda i, te: (te[i], 0, 0),
                         pipeline_mode=pl.Buffered(buffer_count=1)),
        ],
        out_specs=pl.BlockSpec((_R, _D), lambda i, te: (i, 0)),
    )
    return pl.pallas_call(
        _ffn_grouped_body,
        grid_spec=grid_spec,
        out_shape=jax.ShapeDtypeStruct((_NPAD, _D), jnp.float32),
        compiler_params=pltpu.CompilerParams(
            dimension_semantics=("arbitrary",)),
    )(tile_eid, xs, wsrt, up_W, gateproj_W, down_W)


_SR = 256


def _ffn_shared_body(x_ref, up_ref, gp_ref, dn_ref, o_ref):
    xb = x_ref[...]
    u = lax.dot_general(xb, up_ref[...], (((1,), (1,)), ((), ())),
                        preferred_element_type=jnp.float32)
    g = lax.dot_general(xb, gp_ref[...], (((1,), (1,)), ((), ())),
                        preferred_element_type=jnp.float32)
    h = u * jax.nn.sigmoid(u) * g
    o_ref[...] = lax.dot_general(h, dn_ref[...], (((1,), (1,)), ((), ())),
                                 preferred_element_type=jnp.float32)


def _ffn_shared(x, up_s, gateproj_s, down_s):
    return pl.pallas_call(
        _ffn_shared_body,
        grid=(_T // _SR,),
        in_specs=[
            pl.BlockSpec((_SR, _D), lambda i: (i, 0)),
            pl.BlockSpec((_F, _D), lambda i: (0, 0),
                         pipeline_mode=pl.Buffered(buffer_count=1)),
            pl.BlockSpec((_F, _D), lambda i: (0, 0),
                         pipeline_mode=pl.Buffered(buffer_count=1)),
            pl.BlockSpec((_D, _F), lambda i: (0, 0),
                         pipeline_mode=pl.Buffered(buffer_count=1)),
        ],
        out_specs=pl.BlockSpec((_SR, _D), lambda i: (i, 0)),
        out_shape=jax.ShapeDtypeStruct((_T, _D), jnp.float32),
        compiler_params=pltpu.CompilerParams(
            dimension_semantics=("arbitrary",)),
    )(x, up_s, gateproj_s, down_s)


@functools.cache
def _sc_mesh():
    return plsc.VectorSubcoreMesh(core_axis_name="c", subcore_axis_name="s")


_GNB = 3


def _sc_gather_body(x_hbm, perm_hbm, xs_hbm, idx_v,
                    rows0, rows1, rows2, gs0, gs1, gs2, ws0, ws1, ws2):
    wid = lax.axis_index("s") * _NC + lax.axis_index("c")
    base = wid * _BPW
    nch = _BPW // _GCH
    bufs = (rows0, rows1, rows2)
    gsems = (gs0, gs1, gs2)
    wsems = (ws0, ws1, ws2)
    pltpu.sync_copy(perm_hbm.at[pl.ds(base, _BPW)], idx_v)

    def gather(c):
        b = c % _GNB
        return pltpu.async_copy(
            x_hbm.at[idx_v.at[pl.ds(c * _GCH, _GCH)]], bufs[b], gsems[b])

    gcp = [None] * nch
    wcp = [None] * nch
    for c in range(min(_GNB - 1, nch)):
        gcp[c] = gather(c)
    for c in range(nch):
        b = c % _GNB
        if c + _GNB - 1 < nch:
            if c - 1 >= 0:
                wcp[c - 1].wait()
            gcp[c + _GNB - 1] = gather(c + _GNB - 1)
        gcp[c].wait()
        wcp[c] = pltpu.async_copy(
            bufs[b], xs_hbm.at[pl.ds(base + c * _GCH, _GCH)], wsems[b])
    for c in range(max(0, nch - _GNB), nch):
        if wcp[c] is not None:
            wcp[c].wait()


@functools.cache
def _sc_gather():
    return pl.kernel(
        _sc_gather_body,
        mesh=_sc_mesh(),
        out_type=jax.ShapeDtypeStruct((_NPAD, _D), jnp.float32),
        scratch_types=[
            pltpu.VMEM((_BPW,), jnp.int32),
            pltpu.VMEM((_GCH, _D), jnp.float32),
            pltpu.VMEM((_GCH, _D), jnp.float32),
            pltpu.VMEM((_GCH, _D), jnp.float32),
            pltpu.SemaphoreType.DMA,
            pltpu.SemaphoreType.DMA,
            pltpu.SemaphoreType.DMA,
            pltpu.SemaphoreType.DMA,
            pltpu.SemaphoreType.DMA,
            pltpu.SemaphoreType.DMA,
        ],
    )


def _sc_combine_body(o_hbm, sh_hbm, d0_hbm, d1_hbm, y_hbm,
                     idx0, idx1, av0, bv0, sv0, av1, bv1, sv1,
                     gsem0, gsem1, wsem0, wsem1):
    wid = lax.axis_index("s") * _NC + lax.axis_index("c")
    base = wid * _TPW
    nch = _TPW // _CCH
    avs, bvs, svs = (av0, av1), (bv0, bv1), (sv0, sv1)
    gsems = (gsem0, gsem1)
    wsems = (wsem0, wsem1)
    pltpu.sync_copy(d0_hbm.at[pl.ds(base, _TPW)], idx0)
    pltpu.sync_copy(d1_hbm.at[pl.ds(base, _TPW)], idx1)

    def issue(c):
        st = c % 2
        return [
            pltpu.async_copy(o_hbm.at[idx0.at[pl.ds(c * _CCH, _CCH)]],
                             avs[st], gsems[st]),
            pltpu.async_copy(o_hbm.at[idx1.at[pl.ds(c * _CCH, _CCH)]],
                             bvs[st], gsems[st]),
            pltpu.async_copy(sh_hbm.at[pl.ds(base + c * _CCH, _CCH)],
                             svs[st], gsems[st]),
        ]

    pend = [None] * nch
    wpend = [None] * nch
    pend[0] = issue(0)
    for c in range(nch):
        st = c % 2
        if c + 1 < nch:
            if c - 1 >= 0:
                wpend[c - 1].wait()
            pend[c + 1] = issue(c + 1)
        for cp in pend[c]:
            cp.wait()
        av, bv, sv = avs[st], bvs[st], svs[st]

        def row_fn(r, carry):
            for j in range(_D // 16):
                a = av[r, pl.ds(j * 16, 16)]
                b = bv[r, pl.ds(j * 16, 16)]
                s = sv[r, pl.ds(j * 16, 16)]
                av[r, pl.ds(j * 16, 16)] = a + b + s
            return carry

        lax.fori_loop(0, _CCH, row_fn, 0)
        wpend[c] = pltpu.async_copy(
            av, y_hbm.at[pl.ds(base + c * _CCH, _CCH)], wsems[st])
    if nch >= 2:
        wpend[nch - 2].wait()
    wpend[nch - 1].wait()


@functools.cache
def _sc_combine():
    return pl.kernel(
        _sc_combine_body,
        mesh=_sc_mesh(),
        out_type=jax.ShapeDtypeStruct((_T, _D), jnp.float32),
        scratch_types=[
            pltpu.VMEM((_TPW,), jnp.int32),
            pltpu.VMEM((_TPW,), jnp.int32),
            pltpu.VMEM((_CCH, _D), jnp.float32),
            pltpu.VMEM((_CCH, _D), jnp.float32),
            pltpu.VMEM((_CCH, _D), jnp.float32),
            pltpu.VMEM((_CCH, _D), jnp.float32),
            pltpu.VMEM((_CCH, _D), jnp.float32),
            pltpu.VMEM((_CCH, _D), jnp.float32),
            pltpu.SemaphoreType.DMA,
            pltpu.SemaphoreType.DMA,
            pltpu.SemaphoreType.DMA,
            pltpu.SemaphoreType.DMA,
        ],
    )


def kernel(hidden_state, gate_weight, up_W, gateproj_W, down_W,
           up_s, gateproj_s, down_s):
    bsz, sl, dm = hidden_state.shape
    x = hidden_state.reshape(_T, _D)
    pw, dest, te = _route(x, gate_weight)
    perm = pw[:, 0].astype(jnp.int32)
    wsrt = pw[:, 1:2]
    d0 = dest[0]
    d1 = dest[1]
    tile_eid = te.reshape(128)[:_NT]
    xs = _sc_gather()(x, perm)
    o = _ffn_grouped(tile_eid, xs, wsrt, up_W, gateproj_W, down_W)
    sh = _ffn_shared(x, up_s, gateproj_s, down_s)
    y = _sc_combine()(o, sh, d0, d1)
    return y.reshape(bsz, sl, dm)
